# baseline (speedup 1.0000x reference)
import functools

import jax
import jax.numpy as jnp
from jax import lax
from jax.experimental import pallas as pl
from jax.experimental.pallas import tpu as pltpu
from jax.experimental.pallas import tpu_sc as plsc

NC = 2
NS = 16
LANES = 16

SB = 16
NROW_B = 8
DH = 32


def _ceil_to(v, m):
    return (v + m - 1) // m * m


@functools.partial(jax.jit, static_argnames=("n",))
def _seg_sc(x_lo, x_hi, src_p, dst_p, *, n):
    d = x_lo.shape[1]
    share = _ceil_to((n + 1 + NS - 1) // NS, 16)
    rows_tab = NS * share
    nsb = src_p.shape[0] // (SB * NS)
    rows_half = dst_p.shape[0] // NC
    nblk = rows_half // (NROW_B * NS)
    full = n // share
    rem = n - full * share

    sum_t = jax.ShapeDtypeStruct((n, d), jnp.float32)
    cnt_t = jax.ShapeDtypeStruct((n, d), jnp.float32)

    @functools.partial(
        pl.kernel,
        out_type=[sum_t, sum_t, cnt_t, cnt_t],
        mesh=plsc.VectorSubcoreMesh(core_axis_name="c", subcore_axis_name="s"),
        scratch_types=[
            pltpu.VMEM((SB, 128), jnp.int32),
            pltpu.VMEM((SB, 128), jnp.int32),
            pltpu.VMEM((3 * 128, d), jnp.float32),
            pltpu.VMEM((16, d), jnp.float32),
            pltpu.VMEM_SHARED((rows_tab, d), jnp.float32),
            pltpu.SemaphoreType.DMA,
            pltpu.SemaphoreType.DMA,
            pltpu.SemaphoreType.DMA,
            pltpu.SemaphoreType.DMA,
            pltpu.SemaphoreType.DMA,
            pltpu.SemaphoreType.DMA,
        ],
        compiler_params=pltpu.CompilerParams(use_tc_tiling_on_sc=False),
    )
    def seg_kernel(xlo_hbm, xhi_hbm, src_hbm, dst_hbm,
                   sumlo_hbm, sumhi_hbm, cnt0_hbm, cnt1_hbm,
                   srcv, dstv, rowbuf, zb, tab_sh,
                   sg0, sg1, sg2, ss0, ss1, ss2):
        sem_g = (sg0, sg1, sg2)
        sem_s = (ss0, ss1, ss2)
        c = lax.axis_index("c")
        s = lax.axis_index("s")

        zero16 = jnp.zeros((LANES,), jnp.float32)
        one16 = jnp.ones((LANES,), jnp.float32)

        @pl.loop(0, 16)
        def _(r):
            for k in range(d // LANES):
                zb[r, pl.ds(k * LANES, LANES)] = zero16

        row0 = s * share

        def zero_share():
            @pl.loop(0, share // 16)
            def _(t):
                pltpu.sync_copy(zb, tab_sh.at[pl.ds(row0 + t * 16, 16)])

        def drain(out_hbm):
            @pl.when(s < full)
            def _():
                pltpu.sync_copy(tab_sh.at[pl.ds(row0, share)],
                                out_hbm.at[pl.ds(row0, share)])

            if rem > 0:
                @pl.when(s == full)
                def _():
                    pltpu.sync_copy(tab_sh.at[pl.ds(full * share, rem)],
                                    out_hbm.at[pl.ds(full * share, rem)])

        zero_share()
        plsc.subcore_barrier()

        def edge_loop(x_hbm):
            @pl.loop(0, nsb)
            def _(p):
                irow = (s * nsb + p) * SB
                pltpu.sync_copy(src_hbm.at[pl.ds(irow, SB)], srcv)
                pltpu.sync_copy(dst_hbm.at[pl.ds(irow, SB)], dstv)

                gat = [None] * SB
                sca = [None] * SB
                for j in range(SB):
                    if j >= 3:
                        sca[j - 3].wait()
                    gat[j] = pltpu.async_copy(
                        x_hbm.at[srcv.at[j]],
                        rowbuf.at[pl.ds((j % 3) * 128, 128)], sem_g[j % 3])
                    if j >= 1:
                        gat[j - 1].wait()
                        sca[j - 1] = pltpu.async_copy(
                            rowbuf.at[pl.ds(((j - 1) % 3) * 128, 128)],
                            tab_sh.at[dstv.at[j - 1]], sem_s[(j - 1) % 3],
                            add=True)
                gat[SB - 1].wait()
                sca[SB - 1] = pltpu.async_copy(
                    rowbuf.at[pl.ds(((SB - 1) % 3) * 128, 128)],
                    tab_sh.at[dstv.at[SB - 1]], sem_s[(SB - 1) % 3],
                    add=True)
                for j in (SB - 3, SB - 2, SB - 1):
                    sca[j].wait()

        @pl.when(c == 0)
        def _():
            edge_loop(xlo_hbm)

        @pl.when(c == 1)
        def _():
            edge_loop(xhi_hbm)

        plsc.subcore_barrier()

        @pl.when(c == 0)
        def _():
            drain(sumlo_hbm)

        @pl.when(c == 1)
        def _():
            drain(sumhi_hbm)

        zero_share()

        @pl.loop(0, 128)
        def _(r):
            for k in range(d // LANES):
                rowbuf[r, pl.ds(k * LANES, LANES)] = one16

        plsc.subcore_barrier()

        @pl.loop(0, nblk)
        def _(b):
            irow = c * rows_half + (s * nblk + b) * NROW_B
            pltpu.sync_copy(dst_hbm.at[pl.ds(irow, NROW_B)], dstv.at[pl.ds(0, NROW_B)])

            scatters = [
                pltpu.async_copy(rowbuf.at[pl.ds(0, 128)],
                                 tab_sh.at[dstv.at[j]], sem_s[0],
                                 add=True)
                for j in range(NROW_B)
            ]
            for cp in scatters:
                cp.wait()

        plsc.subcore_barrier()

        @pl.when(c == 0)
        def _():
            drain(cnt0_hbm)

        @pl.when(c == 1)
        def _():
            drain(cnt1_hbm)

    return seg_kernel(x_lo, x_hi, src_p, dst_p)


def _dense_body(x_ref, sumlo_ref, sumhi_ref, cnt0_ref, cnt1_ref, wclo_ref,
                wchi_ref, wi_ref, wh_ref, bi_ref, bh_ref, lwi_ref, lb_ref,
                lin_ref, linb_ref, out_ref):
    f32 = jnp.float32
    x = x_ref[...]
    cnt = jnp.maximum(cnt0_ref[:, 0:1] + cnt1_ref[:, 0:1], 1.0)
    agg = (jnp.dot(sumlo_ref[...] / cnt, wclo_ref[...],
                   preferred_element_type=f32)
           + jnp.dot(sumhi_ref[...] / cnt, wchi_ref[...],
                     preferred_element_type=f32))
    gi = jnp.dot(agg, wi_ref[...], preferred_element_type=f32) + bi_ref[...]
    gh = jnp.dot(x, wh_ref[...], preferred_element_type=f32) + bh_ref[...]
    dd = x.shape[1]
    r = jax.nn.sigmoid(gi[:, :dd] + gh[:, :dd])
    z = jax.nn.sigmoid(gi[:, dd:2 * dd] + gh[:, dd:2 * dd])
    nn = jnp.tanh(gi[:, 2 * dd:] + r * gh[:, 2 * dd:])
    h_tilde = (1.0 - z) * nn + z * x
    gates = jnp.dot(h_tilde, lwi_ref[...], preferred_element_type=f32) + lb_ref[...]
    cc = jax.nn.sigmoid(gates[:, :dd]) * jnp.tanh(gates[:, 2 * dd:3 * dd])
    hh = jax.nn.sigmoid(gates[:, 3 * dd:]) * jnp.tanh(cc)
    out_ref[...] = (jnp.dot(jnp.maximum(hh, 0.0), lin_ref[...],
                            preferred_element_type=f32) + linb_ref[...])


def _dense_tc(x, sum_lo, sum_hi, cnt0, cnt1, wc_lo, wc_hi, wi_t, wh_t, bi,
              bh, lwi_t, lb, lin_t, linb, block):
    n, d = x.shape
    grid = n // block

    def rowmap(i):
        return (i, 0)

    def fullmap(i):
        return (0, 0)

    def rspec(width):
        return pl.BlockSpec((block, width), rowmap)

    return pl.pallas_call(
        _dense_body,
        grid=(grid,),
        in_specs=[
            rspec(d), rspec(DH), rspec(DH), rspec(DH), rspec(DH),
            pl.BlockSpec(wc_lo.shape, fullmap),
            pl.BlockSpec(wc_hi.shape, fullmap),
            pl.BlockSpec(wi_t.shape, fullmap),
            pl.BlockSpec(wh_t.shape, fullmap),
            pl.BlockSpec(bi.shape, fullmap),
            pl.BlockSpec(bh.shape, fullmap),
            pl.BlockSpec(lwi_t.shape, fullmap),
            pl.BlockSpec(lb.shape, fullmap),
            pl.BlockSpec(lin_t.shape, fullmap),
            pl.BlockSpec(linb.shape, fullmap),
        ],
        out_specs=pl.BlockSpec((block, d), rowmap),
        out_shape=jax.ShapeDtypeStruct((n, d), jnp.float32),
    )(x, sum_lo, sum_hi, cnt0, cnt1, wc_lo, wc_hi, wi_t, wh_t, bi, bh,
      lwi_t, lb, lin_t, linb)


def kernel(x, edge_index, W_conv, gru_Wi, gru_Wh, gru_bi, gru_bh,
           lstm_Wi, lstm_Wh, lstm_bi, lstm_bh, lin_W, lin_b):
    del lstm_Wh
    n, d = x.shape
    e = edge_index.shape[1]

    src = edge_index[0]
    dst = edge_index[1]

    def pad_to(v, e_pad, fill):
        if e_pad > e:
            v = jnp.concatenate([v, jnp.full((e_pad - e,), fill, jnp.int32)])
        return v.reshape(-1, 128)

    e_pad = _ceil_to(e, max(NS * SB * 128, NC * NS * NROW_B * 128))
    x_lo = x[:, :DH]
    x_hi = x[:, DH:]
    sum_lo, sum_hi, cnt0, cnt1 = _seg_sc(
        x_lo, x_hi, pad_to(src, e_pad, 0), pad_to(dst, e_pad, n), n=n)

    out = _dense_tc(
        x, sum_lo, sum_hi, cnt0, cnt1,
        W_conv[:DH], W_conv[DH:],
        gru_Wi.T, gru_Wh.T, gru_bi[None, :], gru_bh[None, :],
        lstm_Wi.T, (lstm_bi + lstm_bh)[None, :],
        lin_W.T, lin_b[None, :], block=1000)
    return out


# speedup vs baseline: 1.2163x; 1.2163x over previous
import functools

import jax
import jax.numpy as jnp
from jax import lax
from jax.experimental import pallas as pl
from jax.experimental.pallas import tpu as pltpu
from jax.experimental.pallas import tpu_sc as plsc

NC = 2
NS = 16
LANES = 16

SB = 50
NROW_B = 8
DH = 32


def _ceil_to(v, m):
    return (v + m - 1) // m * m


@functools.partial(jax.jit, static_argnames=("n",))
def _seg_vals_sc(x_lo, x_hi, src_p, dst_p, *, n):
    d = x_lo.shape[1]
    share = _ceil_to((n + 1 + NS - 1) // NS, 16)
    rows_tab = NS * share
    nsb = src_p.shape[0] // (SB * NS)
    full = n // share
    rem = n - full * share

    out_t = jax.ShapeDtypeStruct((n, d), jnp.float32)

    @functools.partial(
        pl.kernel,
        out_type=[out_t, out_t],
        mesh=plsc.VectorSubcoreMesh(core_axis_name="c", subcore_axis_name="s"),
        scratch_types=[
            pltpu.VMEM((SB, 128), jnp.int32),
            pltpu.VMEM((SB, 128), jnp.int32),
            pltpu.VMEM((3 * 128, d), jnp.float32),
            pltpu.VMEM((16, d), jnp.float32),
            pltpu.VMEM_SHARED((rows_tab, d), jnp.float32),
            pltpu.SemaphoreType.DMA,
            pltpu.SemaphoreType.DMA,
            pltpu.SemaphoreType.DMA,
            pltpu.SemaphoreType.DMA,
            pltpu.SemaphoreType.DMA,
            pltpu.SemaphoreType.DMA,
        ],
        compiler_params=pltpu.CompilerParams(use_tc_tiling_on_sc=False),
    )
    def seg_kernel(xlo_hbm, xhi_hbm, src_hbm, dst_hbm, sumlo_hbm, sumhi_hbm,
                   srcv, dstv, rowbuf, zb, vals_sh,
                   sg0, sg1, sg2, ss0, ss1, ss2):
        sem_g = (sg0, sg1, sg2)
        sem_s = (ss0, ss1, ss2)
        c = lax.axis_index("c")
        s = lax.axis_index("s")

        zero16 = jnp.zeros((LANES,), jnp.float32)

        @pl.loop(0, 16)
        def _(r):
            for k in range(d // LANES):
                zb[r, pl.ds(k * LANES, LANES)] = zero16

        row0 = s * share

        @pl.loop(0, share // 16)
        def _(t):
            pltpu.sync_copy(zb, vals_sh.at[pl.ds(row0 + t * 16, 16)])

        plsc.subcore_barrier()

        def edge_loop(x_hbm):
            @pl.loop(0, nsb)
            def _(p):
                irow = (s * nsb + p) * SB
                pltpu.sync_copy(src_hbm.at[pl.ds(irow, SB)], srcv)
                pltpu.sync_copy(dst_hbm.at[pl.ds(irow, SB)], dstv)

                gat = [None] * SB
                sca = [None] * SB
                for j in range(SB):
                    if j >= 3:
                        sca[j - 3].wait()
                    gat[j] = pltpu.async_copy(
                        x_hbm.at[srcv.at[j]],
                        rowbuf.at[pl.ds((j % 3) * 128, 128)], sem_g[j % 3])
                    if j >= 1:
                        gat[j - 1].wait()
                        sca[j - 1] = pltpu.async_copy(
                            rowbuf.at[pl.ds(((j - 1) % 3) * 128, 128)],
                            vals_sh.at[dstv.at[j - 1]], sem_s[(j - 1) % 3],
                            add=True)
                gat[SB - 1].wait()
                sca[SB - 1] = pltpu.async_copy(
                    rowbuf.at[pl.ds(((SB - 1) % 3) * 128, 128)],
                    vals_sh.at[dstv.at[SB - 1]], sem_s[(SB - 1) % 3],
                    add=True)
                for j in (SB - 3, SB - 2, SB - 1):
                    sca[j].wait()

        @pl.when(c == 0)
        def _():
            edge_loop(xlo_hbm)

        @pl.when(c == 1)
        def _():
            edge_loop(xhi_hbm)

        plsc.subcore_barrier()

        def drain(sum_hbm):
            @pl.when(s < full)
            def _():
                pltpu.sync_copy(vals_sh.at[pl.ds(row0, share)],
                                sum_hbm.at[pl.ds(row0, share)])

            if rem > 0:
                @pl.when(s == full)
                def _():
                    pltpu.sync_copy(vals_sh.at[pl.ds(full * share, rem)],
                                    sum_hbm.at[pl.ds(full * share, rem)])

        @pl.when(c == 0)
        def _():
            drain(sumlo_hbm)

        @pl.when(c == 1)
        def _():
            drain(sumhi_hbm)

    return seg_kernel(x_lo, x_hi, src_p, dst_p)


@functools.partial(jax.jit, static_argnames=("n",))
def _seg_cnts_sc(dst_p, *, n):
    share = _ceil_to((n + 1 + NS - 1) // NS, 16)
    rows_tab = NS * share
    rows_half = dst_p.shape[0] // NC
    nblk = rows_half // (NROW_B * NS)
    full = n // share
    rem = n - full * share

    out_t = jax.ShapeDtypeStruct((n, LANES), jnp.float32)

    @functools.partial(
        pl.kernel,
        out_type=[out_t, out_t],
        mesh=plsc.VectorSubcoreMesh(core_axis_name="c", subcore_axis_name="s"),
        scratch_types=[
            pltpu.VMEM((NROW_B, 128), jnp.int32),
            pltpu.VMEM((128, LANES), jnp.float32),
            pltpu.VMEM((16, LANES), jnp.float32),
            pltpu.VMEM_SHARED((rows_tab, LANES), jnp.float32),
            pltpu.SemaphoreType.DMA,
        ],
        compiler_params=pltpu.CompilerParams(use_tc_tiling_on_sc=False),
    )
    def cnt_kernel(dst_hbm, cnt0_hbm, cnt1_hbm, dstv, ones_v, zbc, cnts_sh,
                   sem_s):
        c = lax.axis_index("c")
        s = lax.axis_index("s")

        zero16 = jnp.zeros((LANES,), jnp.float32)
        one16 = jnp.ones((LANES,), jnp.float32)

        @pl.loop(0, 16)
        def _(r):
            zbc[r, pl.ds(0, LANES)] = zero16

        @pl.loop(0, 128)
        def _(r):
            ones_v[r, pl.ds(0, LANES)] = one16

        row0 = s * share

        @pl.loop(0, share // 16)
        def _(t):
            pltpu.sync_copy(zbc, cnts_sh.at[pl.ds(row0 + t * 16, 16)])

        plsc.subcore_barrier()

        @pl.loop(0, nblk)
        def _(b):
            irow = c * rows_half + (s * nblk + b) * NROW_B
            pltpu.sync_copy(dst_hbm.at[pl.ds(irow, NROW_B)], dstv)

            scatters = [
                pltpu.async_copy(ones_v, cnts_sh.at[dstv.at[j]], sem_s,
                                 add=True)
                for j in range(NROW_B)
            ]
            for cp in scatters:
                cp.wait()

        plsc.subcore_barrier()

        def drain(cnt_hbm):
            @pl.when(s < full)
            def _():
                pltpu.sync_copy(cnts_sh.at[pl.ds(row0, share)],
                                cnt_hbm.at[pl.ds(row0, share)])

            if rem > 0:
                @pl.when(s == full)
                def _():
                    pltpu.sync_copy(cnts_sh.at[pl.ds(full * share, rem)],
                                    cnt_hbm.at[pl.ds(full * share, rem)])

        @pl.when(c == 0)
        def _():
            drain(cnt0_hbm)

        @pl.when(c == 1)
        def _():
            drain(cnt1_hbm)

    return cnt_kernel(dst_p)


def _dense_body(x_ref, sumlo_ref, sumhi_ref, cnt0_ref, cnt1_ref, wclo_ref,
                wchi_ref, wi_ref, wh_ref, bi_ref, bh_ref, lwi_ref, lb_ref,
                lin_ref, linb_ref, out_ref):
    f32 = jnp.float32
    x = x_ref[...]
    cnt = jnp.maximum(cnt0_ref[:, 0:1] + cnt1_ref[:, 0:1], 1.0)
    agg = (jnp.dot(sumlo_ref[...] / cnt, wclo_ref[...],
                   preferred_element_type=f32)
           + jnp.dot(sumhi_ref[...] / cnt, wchi_ref[...],
                     preferred_element_type=f32))
    gi = jnp.dot(agg, wi_ref[...], preferred_element_type=f32) + bi_ref[...]
    gh = jnp.dot(x, wh_ref[...], preferred_element_type=f32) + bh_ref[...]
    dd = x.shape[1]
    r = jax.nn.sigmoid(gi[:, :dd] + gh[:, :dd])
    z = jax.nn.sigmoid(gi[:, dd:2 * dd] + gh[:, dd:2 * dd])
    nn = jnp.tanh(gi[:, 2 * dd:] + r * gh[:, 2 * dd:])
    h_tilde = (1.0 - z) * nn + z * x
    gates = jnp.dot(h_tilde, lwi_ref[...], preferred_element_type=f32) + lb_ref[...]
    cc = jax.nn.sigmoid(gates[:, :dd]) * jnp.tanh(gates[:, 2 * dd:3 * dd])
    hh = jax.nn.sigmoid(gates[:, 3 * dd:]) * jnp.tanh(cc)
    out_ref[...] = (jnp.dot(jnp.maximum(hh, 0.0), lin_ref[...],
                            preferred_element_type=f32) + linb_ref[...])


def _dense_tc(x, sum_lo, sum_hi, cnt0, cnt1, wc_lo, wc_hi, wi_t, wh_t, bi,
              bh, lwi_t, lb, lin_t, linb, block):
    n, d = x.shape
    grid = n // block

    def rowmap(i):
        return (i, 0)

    def fullmap(i):
        return (0, 0)

    def rspec(width):
        return pl.BlockSpec((block, width), rowmap)

    return pl.pallas_call(
        _dense_body,
        grid=(grid,),
        in_specs=[
            rspec(d), rspec(DH), rspec(DH), rspec(LANES), rspec(LANES),
            pl.BlockSpec(wc_lo.shape, fullmap),
            pl.BlockSpec(wc_hi.shape, fullmap),
            pl.BlockSpec(wi_t.shape, fullmap),
            pl.BlockSpec(wh_t.shape, fullmap),
            pl.BlockSpec(bi.shape, fullmap),
            pl.BlockSpec(bh.shape, fullmap),
            pl.BlockSpec(lwi_t.shape, fullmap),
            pl.BlockSpec(lb.shape, fullmap),
            pl.BlockSpec(lin_t.shape, fullmap),
            pl.BlockSpec(linb.shape, fullmap),
        ],
        out_specs=pl.BlockSpec((block, d), rowmap),
        out_shape=jax.ShapeDtypeStruct((n, d), jnp.float32),
    )(x, sum_lo, sum_hi, cnt0, cnt1, wc_lo, wc_hi, wi_t, wh_t, bi, bh,
      lwi_t, lb, lin_t, linb)


def kernel(x, edge_index, W_conv, gru_Wi, gru_Wh, gru_bi, gru_bh,
           lstm_Wi, lstm_Wh, lstm_bi, lstm_bh, lin_W, lin_b):
    del lstm_Wh
    n, d = x.shape
    e = edge_index.shape[1]

    src = edge_index[0]
    dst = edge_index[1]

    def pad_to(v, e_pad, fill):
        if e_pad > e:
            v = jnp.concatenate([v, jnp.full((e_pad - e,), fill, jnp.int32)])
        return v.reshape(-1, 128)

    e_pad_a = _ceil_to(e, NS * SB * 128)
    e_pad_b = _ceil_to(e, NC * NS * NROW_B * 128)
    x_lo = x[:, :DH]
    x_hi = x[:, DH:]
    sum_lo, sum_hi = _seg_vals_sc(x_lo, x_hi, pad_to(src, e_pad_a, 0),
                                  pad_to(dst, e_pad_a, n), n=n)
    cnt0, cnt1 = _seg_cnts_sc(pad_to(dst, e_pad_b, n), n=n)

    out = _dense_tc(
        x, sum_lo, sum_hi, cnt0, cnt1,
        W_conv[:DH], W_conv[DH:],
        gru_Wi.T, gru_Wh.T, gru_bi[None, :], gru_bh[None, :],
        lstm_Wi.T, (lstm_bi + lstm_bh)[None, :],
        lin_W.T, lin_b[None, :], block=2000)
    return out
